# baseline (speedup 1.0000x reference)
import jax
import jax.numpy as jnp
from jax import lax
from jax.experimental import pallas as pl
from jax.experimental.pallas import tpu as pltpu
from jax.experimental.pallas import tpu_sc as plsc

N = 10000
E = 320000
D = 128
H = 128
K_HOPS = 3

NC = 2
NS = 16
NW = NC * NS
LANES = 16

N_PAD = 10240
ROWS_PER_TILE = N_PAD // NS
W_ROWS = N_PAD // NW
ROW_CHUNK = 64
DEG_CHUNK = 128
N_DEG_CHUNKS = ROWS_PER_TILE // DEG_CHUNK

EDGE_CHUNK = 128
BATCH = 8
BATCH_E = BATCH * EDGE_CHUNK
B_FAST = 20
B_SLOW = 0
N_BATCHES_PAIR = B_FAST + B_SLOW
E_PAD = N_BATCHES_PAIR * BATCH_E * NS

_MESH = plsc.VectorSubcoreMesh(core_axis_name="c", subcore_axis_name="s")


def _rsqrt16(x):
    i = lax.bitcast_convert_type(x, jnp.int32)
    i = jnp.int32(0x5F3759DF) - lax.shift_right_arithmetic(i, 1)
    y = lax.bitcast_convert_type(i, jnp.float32)
    for _ in range(3):
        y = y * (jnp.float32(1.5) - jnp.float32(0.5) * x * y * y)
    return y


def _worker(c, s):
    return c * NS + s


def _deg_body(dst2_hbm, pdeg0, pdeg1,
              degstage_sh, deg_v, degred_v, idx_d):
    c = lax.axis_index("c")
    s = lax.axis_index("s")
    w = _worker(c, s)
    r0 = s * ROWS_PER_TILE
    dbase = w * ((E_PAD // NW) // EDGE_CHUNK)

    def zero_deg(i, _):
        deg_v[pl.ds(i * LANES, LANES)] = jnp.zeros((LANES,), jnp.float32)
        return _
    lax.fori_loop(0, N_PAD // LANES, zero_deg, 0)

    ones = jnp.ones((LANES,), jnp.float32)

    def deg_batch(ib, _):
        pltpu.sync_copy(dst2_hbm.at[pl.ds(dbase + ib * BATCH, BATCH)], idx_d)
        def deg_row(t, _):
            for q in range(EDGE_CHUNK // LANES):
                idx = idx_d[t, pl.ds(q * LANES, LANES)]
                plsc.addupdate_scatter(deg_v, [idx], ones)
            return _
        lax.fori_loop(0, BATCH, deg_row, 0)
        return _
    lax.fori_loop(0, E_PAD // NW // BATCH_E, deg_batch, 0)

    pltpu.sync_copy(deg_v, degstage_sh.at[s])
    plsc.subcore_barrier()

    def deg_reduce(j, _):
        pltpu.sync_copy(degstage_sh.at[:, pl.ds(r0 + j * DEG_CHUNK,
                                                DEG_CHUNK)], degred_v)
        for q in range(DEG_CHUNK // LANES):
            acc = jnp.zeros((LANES,), jnp.float32)
            for k in range(NS):
                acc = acc + degred_v[k, pl.ds(q * LANES, LANES)]
            deg_v[pl.ds(j * DEG_CHUNK + q * LANES, LANES)] = acc
        return _
    lax.fori_loop(0, N_DEG_CHUNKS, deg_reduce, 0)

    @pl.when(c == 0)
    def _dump0():
        pltpu.sync_copy(deg_v.at[pl.ds(0, ROWS_PER_TILE)],
                        pdeg0.at[pl.ds(r0, ROWS_PER_TILE)])

    @pl.when(c == 1)
    def _dump1():
        pltpu.sync_copy(deg_v.at[pl.ds(0, ROWS_PER_TILE)],
                        pdeg1.at[pl.ds(r0, ROWS_PER_TILE)])


def _init_body(x_hbm, pdeg0, pdeg1,
               dinv_out, dinv2_out, g0_out,
               d0_v, d1_v, dv_v, dv2_v, xbuf):
    c = lax.axis_index("c")
    s = lax.axis_index("s")
    r0 = _worker(c, s) * W_ROWS

    pltpu.sync_copy(pdeg0.at[pl.ds(r0, W_ROWS)], d0_v)
    pltpu.sync_copy(pdeg1.at[pl.ds(r0, W_ROWS)], d1_v)

    def dv_group(j, _):
        cs = pl.ds(j * LANES, LANES)
        dtot = d0_v[cs] + d1_v[cs] + jnp.float32(1.0)
        y = _rsqrt16(dtot)
        dv_v[cs] = y
        dv2_v[cs] = y * y
        return _
    lax.fori_loop(0, W_ROWS // LANES, dv_group, 0)

    pltpu.sync_copy(dv_v, dinv_out.at[pl.ds(r0, W_ROWS)])
    pltpu.sync_copy(dv2_v, dinv2_out.at[pl.ds(r0, W_ROWS)])

    def init_chunk(j, _):
        r = r0 + j * ROW_CHUNK
        pltpu.sync_copy(x_hbm.at[pl.ds(r, ROW_CHUNK)], xbuf)
        def g16_grp(g16, _):
            dvc = dv_v[pl.ds(j * ROW_CHUNK + g16 * LANES, LANES)]
            for lane in range(LANES):
                sc = dvc[lane]
                for q in range(D // LANES):
                    cs = pl.ds(q * LANES, LANES)
                    xbuf[g16 * LANES + lane, cs] = (
                        xbuf[g16 * LANES + lane, cs] * sc)
            return _
        lax.fori_loop(0, ROW_CHUNK // LANES, g16_grp, 0)
        pltpu.sync_copy(xbuf, g0_out.at[pl.ds(r, ROW_CHUNK)])
        return _
    lax.fori_loop(0, W_ROWS // ROW_CHUNK, init_chunk, 0)


def _edge_body(g_hbm, src_hbm, dst2_hbm,
               part0,
               acc_sh, idx_s, idx_d, rows0, rows1,
               sem_g0, sem_g1, sem_s0, sem_s1):
    c = lax.axis_index("c")
    s = lax.axis_index("s")
    slab = s * ROWS_PER_TILE
    nb = jnp.where(c == 0, B_FAST, B_SLOW)
    bbase = jnp.where(c == 0, s * B_FAST, NS * B_FAST + s * B_SLOW)
    ebase = bbase * BATCH_E
    dbase = bbase * BATCH

    @pl.when(nb > 0)
    def _zero_acc():
        def zrow(i, _):
            for q in range(D // LANES):
                rows0[i, pl.ds(q * LANES, LANES)] = jnp.zeros((LANES,),
                                                              jnp.float32)
            return _
        lax.fori_loop(0, EDGE_CHUNK, zrow, 0)
        def zcopy(i, _):
            pltpu.sync_copy(rows0,
                            acc_sh.at[pl.ds(slab + i * EDGE_CHUNK,
                                            EDGE_CHUNK)])
            return _
        lax.fori_loop(0, ROWS_PER_TILE // EDGE_CHUNK, zcopy, 0)
    plsc.subcore_barrier()

    def g_start(buf, sem, j):
        pltpu.async_copy(
            g_hbm.at[idx_s.at[pl.ds(j * EDGE_CHUNK, EDGE_CHUNK)]], buf, sem)

    def g_wait(buf, sem):
        pltpu.make_async_copy(
            g_hbm.at[idx_s.at[pl.ds(0, EDGE_CHUNK)]], buf, sem).wait()

    def s_start(buf, sem, j):
        pltpu.async_copy(buf, acc_sh.at[idx_d.at[j]], sem, add=True)

    def s_wait(buf, sem):
        pltpu.make_async_copy(buf, acc_sh.at[idx_d.at[0]], sem).wait()

    def emit_batch(ib, _):
        pltpu.sync_copy(src_hbm.at[pl.ds(ebase + ib * BATCH_E, BATCH_E)],
                        idx_s)
        pltpu.sync_copy(dst2_hbm.at[pl.ds(dbase + ib * BATCH, BATCH)], idx_d)
        g_start(rows0, sem_g0, 0)
        g_wait(rows0, sem_g0)

        @pl.when(ib > 0)
        def _drain_prev():
            s_wait(rows1, sem_s1)
        s_start(rows0, sem_s0, 0)
        g_start(rows1, sem_g1, 1)

        def pair(k, _):
            c1 = 2 * k + 1
            c2 = 2 * k + 2
            g_wait(rows1, sem_g1)
            s_wait(rows0, sem_s0)
            s_start(rows1, sem_s1, c1)
            g_start(rows0, sem_g0, c2)
            g_wait(rows0, sem_g0)
            s_wait(rows1, sem_s1)
            s_start(rows0, sem_s0, c2)
            g_start(rows1, sem_g1, c2 + 1)
            return _
        lax.fori_loop(0, BATCH // 2 - 1, pair, 0)

        g_wait(rows1, sem_g1)
        s_wait(rows0, sem_s0)
        s_start(rows1, sem_s1, BATCH - 1)
        return _
    lax.fori_loop(0, nb, emit_batch, 0)

    @pl.when(nb > 0)
    def _drain_last():
        s_wait(rows1, sem_s1)
    plsc.subcore_barrier()

    @pl.when(nb > 0)
    def _dump0():
        pltpu.sync_copy(acc_sh.at[pl.ds(slab, ROWS_PER_TILE)],
                        part0.at[pl.ds(slab, ROWS_PER_TILE)])


def _rescale_body(p0_hbm, g_hbm, scale_hbm,
                  out_hbm,
                  b0, b2, sc_v):
    c = lax.axis_index("c")
    s = lax.axis_index("s")
    r0 = _worker(c, s) * W_ROWS

    pltpu.sync_copy(scale_hbm.at[pl.ds(r0, W_ROWS)], sc_v)

    def resc_chunk(j, _):
        r = r0 + j * ROW_CHUNK
        pltpu.sync_copy(p0_hbm.at[pl.ds(r, ROW_CHUNK)], b0)
        pltpu.sync_copy(g_hbm.at[pl.ds(r, ROW_CHUNK)], b2)
        def g16_grp(g16, _):
            dvc = sc_v[pl.ds(j * ROW_CHUNK + g16 * LANES, LANES)]
            for lane in range(LANES):
                sc = dvc[lane]
                rr = g16 * LANES + lane
                for q in range(D // LANES):
                    cs = pl.ds(q * LANES, LANES)
                    b2[rr, cs] = (b2[rr, cs] + b0[rr, cs]) * sc
            return _
        lax.fori_loop(0, ROW_CHUNK // LANES, g16_grp, 0)
        pltpu.sync_copy(b2, out_hbm.at[pl.ds(r, ROW_CHUNK)])
        return _
    lax.fori_loop(0, W_ROWS // ROW_CHUNK, resc_chunk, 0)


_deg_k = pl.kernel(
    _deg_body,
    out_type=(
        jax.ShapeDtypeStruct((N_PAD,), jnp.float32),
        jax.ShapeDtypeStruct((N_PAD,), jnp.float32),
    ),
    mesh=_MESH,
    scratch_types=[
        pltpu.VMEM_SHARED((NS, N_PAD), jnp.float32),
        pltpu.VMEM((N_PAD,), jnp.float32),
        pltpu.VMEM((NS, DEG_CHUNK), jnp.float32),
        pltpu.VMEM((BATCH, EDGE_CHUNK), jnp.int32),
    ],
    compiler_params=pltpu.CompilerParams(needs_layout_passes=False),
)

_init_k = pl.kernel(
    _init_body,
    out_type=(
        jax.ShapeDtypeStruct((N_PAD,), jnp.float32),
        jax.ShapeDtypeStruct((N_PAD,), jnp.float32),
        jax.ShapeDtypeStruct((N_PAD, D), jnp.float32),
    ),
    mesh=_MESH,
    scratch_types=[
        pltpu.VMEM((W_ROWS,), jnp.float32),
        pltpu.VMEM((W_ROWS,), jnp.float32),
        pltpu.VMEM((W_ROWS,), jnp.float32),
        pltpu.VMEM((W_ROWS,), jnp.float32),
        pltpu.VMEM((ROW_CHUNK, D), jnp.float32),
    ],
    compiler_params=pltpu.CompilerParams(needs_layout_passes=False),
)

_edge_k = pl.kernel(
    _edge_body,
    out_type=jax.ShapeDtypeStruct((N_PAD, D), jnp.float32),

    mesh=_MESH,
    scratch_types=[
        pltpu.VMEM_SHARED((N_PAD, D), jnp.float32),
        pltpu.VMEM((BATCH_E,), jnp.int32),
        pltpu.VMEM((BATCH, EDGE_CHUNK), jnp.int32),
        pltpu.VMEM((EDGE_CHUNK, D), jnp.float32),
        pltpu.VMEM((EDGE_CHUNK, D), jnp.float32),
        pltpu.SemaphoreType.DMA,
        pltpu.SemaphoreType.DMA,
        pltpu.SemaphoreType.DMA,
        pltpu.SemaphoreType.DMA,
    ],
    compiler_params=pltpu.CompilerParams(needs_layout_passes=False),
)

_rescale_k = pl.kernel(
    _rescale_body,
    out_type=jax.ShapeDtypeStruct((N_PAD, D), jnp.float32),
    mesh=_MESH,
    scratch_types=[
        pltpu.VMEM((ROW_CHUNK, D), jnp.float32),
        pltpu.VMEM((ROW_CHUNK, D), jnp.float32),
        pltpu.VMEM((W_ROWS,), jnp.float32),
    ],
    compiler_params=pltpu.CompilerParams(needs_layout_passes=False),
)


@jax.jit
def _propagate(x_pad, src_pad, dst2_pad):
    pdeg0, pdeg1 = _deg_k(dst2_pad)
    dinv, dinv2, g = _init_k(x_pad, pdeg0, pdeg1)
    for hop in range(K_HOPS):
        part0 = _edge_k(g, src_pad, dst2_pad)
        scale = dinv if hop == K_HOPS - 1 else dinv2
        g = _rescale_k(part0, g, scale)
    return g


def _linear_elu_body(h_ref, wt_ref, b_ref, o_ref):
    o = (jnp.dot(h_ref[:], wt_ref[:], preferred_element_type=jnp.float32)
         + b_ref[:])
    o_ref[:, :] = jnp.where(o > 0, o, jnp.exp(o) - jnp.float32(1.0))


@jax.jit
def _linear_elu(h2, wt, b2d):
    blk = 1024
    grid = (N_PAD // blk,)
    return pl.pallas_call(
        _linear_elu_body,
        grid=grid,
        in_specs=[
            pl.BlockSpec((blk, D), lambda i: (i, 0)),
            pl.BlockSpec((D, H), lambda i: (0, 0)),
            pl.BlockSpec((1, H), lambda i: (0, 0)),
        ],
        out_specs=pl.BlockSpec((blk, H), lambda i: (i, 0)),
        out_shape=jax.ShapeDtypeStruct((N_PAD, H), jnp.float32),
    )(h2, wt, b2d)


def kernel(x, edge_index, W, b):
    x_pad = jnp.zeros((N_PAD, D), jnp.float32).at[:N].set(x)
    pad_idx = jnp.full((E_PAD - E,), N_PAD - 1, jnp.int32)
    src_pad = jnp.concatenate([edge_index[0], pad_idx])
    dst_pad = jnp.concatenate([edge_index[1], pad_idx])
    dst2_pad = dst_pad.reshape(E_PAD // EDGE_CHUNK, EDGE_CHUNK)
    h2 = _propagate(x_pad, src_pad, dst2_pad)

    out = _linear_elu(h2, W.T, b[None, :])
    return out[:N]


# speedup vs baseline: 1.3430x; 1.3430x over previous
import jax
import jax.numpy as jnp
from jax import lax
from jax.experimental import pallas as pl
from jax.experimental.pallas import tpu as pltpu
from jax.experimental.pallas import tpu_sc as plsc

N = 10000
E = 320000
D = 128
H = 128
K_HOPS = 3

NC = 2
NS = 16
NW = NC * NS
LANES = 16

N_PAD = 10240
ROWS_PER_TILE = N_PAD // NS
W_ROWS = N_PAD // NW
ROW_CHUNK = 64
DEG_CHUNK = 128
N_DEG_CHUNKS = ROWS_PER_TILE // DEG_CHUNK

EDGE_CHUNK = 128
BATCH = 8
BATCH_E = BATCH * EDGE_CHUNK
B_FAST = 15
B_SLOW = 5
N_BATCHES_PAIR = B_FAST + B_SLOW
E_PAD = N_BATCHES_PAIR * BATCH_E * NS

_MESH = plsc.VectorSubcoreMesh(core_axis_name="c", subcore_axis_name="s")


def _rsqrt16(x):
    i = lax.bitcast_convert_type(x, jnp.int32)
    i = jnp.int32(0x5F3759DF) - lax.shift_right_arithmetic(i, 1)
    y = lax.bitcast_convert_type(i, jnp.float32)
    for _ in range(3):
        y = y * (jnp.float32(1.5) - jnp.float32(0.5) * x * y * y)
    return y


def _worker(c, s):
    return c * NS + s


def _deg_body(dst2_hbm, pdeg0, pdeg1,
              degstage_sh, deg_v, degred_v, idx_d):
    c = lax.axis_index("c")
    s = lax.axis_index("s")
    w = _worker(c, s)
    r0 = s * ROWS_PER_TILE
    dbase = w * ((E_PAD // NW) // EDGE_CHUNK)

    def zero_deg(i, _):
        deg_v[pl.ds(i * LANES, LANES)] = jnp.zeros((LANES,), jnp.float32)
        return _
    lax.fori_loop(0, N_PAD // LANES, zero_deg, 0)

    ones = jnp.ones((LANES,), jnp.float32)

    def deg_batch(ib, _):
        pltpu.sync_copy(dst2_hbm.at[pl.ds(dbase + ib * BATCH, BATCH)], idx_d)
        def deg_row(t, _):
            for q in range(EDGE_CHUNK // LANES):
                idx = idx_d[t, pl.ds(q * LANES, LANES)]
                plsc.addupdate_scatter(deg_v, [idx], ones)
            return _
        lax.fori_loop(0, BATCH, deg_row, 0)
        return _
    lax.fori_loop(0, E_PAD // NW // BATCH_E, deg_batch, 0)

    pltpu.sync_copy(deg_v, degstage_sh.at[s])
    plsc.subcore_barrier()

    def deg_reduce(j, _):
        pltpu.sync_copy(degstage_sh.at[:, pl.ds(r0 + j * DEG_CHUNK,
                                                DEG_CHUNK)], degred_v)
        for q in range(DEG_CHUNK // LANES):
            acc = jnp.zeros((LANES,), jnp.float32)
            for k in range(NS):
                acc = acc + degred_v[k, pl.ds(q * LANES, LANES)]
            deg_v[pl.ds(j * DEG_CHUNK + q * LANES, LANES)] = acc
        return _
    lax.fori_loop(0, N_DEG_CHUNKS, deg_reduce, 0)

    @pl.when(c == 0)
    def _dump0():
        pltpu.sync_copy(deg_v.at[pl.ds(0, ROWS_PER_TILE)],
                        pdeg0.at[pl.ds(r0, ROWS_PER_TILE)])

    @pl.when(c == 1)
    def _dump1():
        pltpu.sync_copy(deg_v.at[pl.ds(0, ROWS_PER_TILE)],
                        pdeg1.at[pl.ds(r0, ROWS_PER_TILE)])


def _init_body(x_hbm, pdeg0, pdeg1,
               dinv_out, dinv2_out, g0_out,
               d0_v, d1_v, dv_v, dv2_v, xbuf):
    c = lax.axis_index("c")
    s = lax.axis_index("s")
    r0 = _worker(c, s) * W_ROWS

    pltpu.sync_copy(pdeg0.at[pl.ds(r0, W_ROWS)], d0_v)
    pltpu.sync_copy(pdeg1.at[pl.ds(r0, W_ROWS)], d1_v)

    def dv_group(j, _):
        cs = pl.ds(j * LANES, LANES)
        dtot = d0_v[cs] + d1_v[cs] + jnp.float32(1.0)
        y = _rsqrt16(dtot)
        dv_v[cs] = y
        dv2_v[cs] = y * y
        return _
    lax.fori_loop(0, W_ROWS // LANES, dv_group, 0)

    pltpu.sync_copy(dv_v, dinv_out.at[pl.ds(r0, W_ROWS)])
    pltpu.sync_copy(dv2_v, dinv2_out.at[pl.ds(r0, W_ROWS)])

    def init_chunk(j, _):
        r = r0 + j * ROW_CHUNK
        pltpu.sync_copy(x_hbm.at[pl.ds(r, ROW_CHUNK)], xbuf)
        def g16_grp(g16, _):
            dvc = dv_v[pl.ds(j * ROW_CHUNK + g16 * LANES, LANES)]
            for lane in range(LANES):
                sc = dvc[lane]
                for q in range(D // LANES):
                    cs = pl.ds(q * LANES, LANES)
                    xbuf[g16 * LANES + lane, cs] = (
                        xbuf[g16 * LANES + lane, cs] * sc)
            return _
        lax.fori_loop(0, ROW_CHUNK // LANES, g16_grp, 0)
        pltpu.sync_copy(xbuf, g0_out.at[pl.ds(r, ROW_CHUNK)])
        return _
    lax.fori_loop(0, W_ROWS // ROW_CHUNK, init_chunk, 0)


def _edge_body(g_hbm, src_hbm, dst2_hbm,
               part0, part1,
               acc_sh, idx_s, idx_d, rows0, rows1,
               sem_g0, sem_g1, sem_s0, sem_s1):
    c = lax.axis_index("c")
    s = lax.axis_index("s")
    slab = s * ROWS_PER_TILE
    nb = jnp.where(c == 0, B_FAST, B_SLOW)
    bbase = jnp.where(c == 0, s * B_FAST, NS * B_FAST + s * B_SLOW)
    ebase = bbase * BATCH_E
    dbase = bbase * BATCH

    @pl.when(nb > 0)
    def _zero_acc():
        def zrow(i, _):
            for q in range(D // LANES):
                rows0[i, pl.ds(q * LANES, LANES)] = jnp.zeros((LANES,),
                                                              jnp.float32)
            return _
        lax.fori_loop(0, EDGE_CHUNK, zrow, 0)
        def zcopy(i, _):
            pltpu.sync_copy(rows0,
                            acc_sh.at[pl.ds(slab + i * EDGE_CHUNK,
                                            EDGE_CHUNK)])
            return _
        lax.fori_loop(0, ROWS_PER_TILE // EDGE_CHUNK, zcopy, 0)
    plsc.subcore_barrier()

    def g_start(buf, sem, j):
        pltpu.async_copy(
            g_hbm.at[idx_s.at[pl.ds(j * EDGE_CHUNK, EDGE_CHUNK)]], buf, sem)

    def g_wait(buf, sem):
        pltpu.make_async_copy(
            g_hbm.at[idx_s.at[pl.ds(0, EDGE_CHUNK)]], buf, sem).wait()

    def s_start(buf, sem, j):
        pltpu.async_copy(buf, acc_sh.at[idx_d.at[j]], sem, add=True)

    def s_wait(buf, sem):
        pltpu.make_async_copy(buf, acc_sh.at[idx_d.at[0]], sem).wait()

    def emit_batch(ib, _):
        pltpu.sync_copy(src_hbm.at[pl.ds(ebase + ib * BATCH_E, BATCH_E)],
                        idx_s)
        pltpu.sync_copy(dst2_hbm.at[pl.ds(dbase + ib * BATCH, BATCH)], idx_d)
        g_start(rows0, sem_g0, 0)
        g_wait(rows0, sem_g0)

        @pl.when(ib > 0)
        def _drain_prev():
            s_wait(rows1, sem_s1)
        s_start(rows0, sem_s0, 0)
        g_start(rows1, sem_g1, 1)

        def pair(k, _):
            c1 = 2 * k + 1
            c2 = 2 * k + 2
            g_wait(rows1, sem_g1)
            s_wait(rows0, sem_s0)
            s_start(rows1, sem_s1, c1)
            g_start(rows0, sem_g0, c2)
            g_wait(rows0, sem_g0)
            s_wait(rows1, sem_s1)
            s_start(rows0, sem_s0, c2)
            g_start(rows1, sem_g1, c2 + 1)
            return _
        lax.fori_loop(0, BATCH // 2 - 1, pair, 0)

        g_wait(rows1, sem_g1)
        s_wait(rows0, sem_s0)
        s_start(rows1, sem_s1, BATCH - 1)
        return _
    lax.fori_loop(0, nb, emit_batch, 0)

    @pl.when(nb > 0)
    def _drain_last():
        s_wait(rows1, sem_s1)
    plsc.subcore_barrier()

    @pl.when(c == 0)
    def _dump0():
        pltpu.sync_copy(acc_sh.at[pl.ds(slab, ROWS_PER_TILE)],
                        part0.at[pl.ds(slab, ROWS_PER_TILE)])

    @pl.when(c == 1)
    def _dump1():
        pltpu.sync_copy(acc_sh.at[pl.ds(slab, ROWS_PER_TILE)],
                        part1.at[pl.ds(slab, ROWS_PER_TILE)])


def _rescale_body(p0_hbm, p1_hbm, g_hbm, scale_hbm,
                  out_hbm,
                  b0, b1, b2, sc_v):
    c = lax.axis_index("c")
    s = lax.axis_index("s")
    r0 = _worker(c, s) * W_ROWS

    pltpu.sync_copy(scale_hbm.at[pl.ds(r0, W_ROWS)], sc_v)

    def resc_chunk(j, _):
        r = r0 + j * ROW_CHUNK
        pltpu.sync_copy(p0_hbm.at[pl.ds(r, ROW_CHUNK)], b0)
        pltpu.sync_copy(p1_hbm.at[pl.ds(r, ROW_CHUNK)], b1)
        pltpu.sync_copy(g_hbm.at[pl.ds(r, ROW_CHUNK)], b2)
        def g16_grp(g16, _):
            dvc = sc_v[pl.ds(j * ROW_CHUNK + g16 * LANES, LANES)]
            for lane in range(LANES):
                sc = dvc[lane]
                rr = g16 * LANES + lane
                for q in range(D // LANES):
                    cs = pl.ds(q * LANES, LANES)
                    b2[rr, cs] = (b2[rr, cs] + b0[rr, cs] + b1[rr, cs]) * sc
            return _
        lax.fori_loop(0, ROW_CHUNK // LANES, g16_grp, 0)
        pltpu.sync_copy(b2, out_hbm.at[pl.ds(r, ROW_CHUNK)])
        return _
    lax.fori_loop(0, W_ROWS // ROW_CHUNK, resc_chunk, 0)


_deg_k = pl.kernel(
    _deg_body,
    out_type=(
        jax.ShapeDtypeStruct((N_PAD,), jnp.float32),
        jax.ShapeDtypeStruct((N_PAD,), jnp.float32),
    ),
    mesh=_MESH,
    scratch_types=[
        pltpu.VMEM_SHARED((NS, N_PAD), jnp.float32),
        pltpu.VMEM((N_PAD,), jnp.float32),
        pltpu.VMEM((NS, DEG_CHUNK), jnp.float32),
        pltpu.VMEM((BATCH, EDGE_CHUNK), jnp.int32),
    ],
    compiler_params=pltpu.CompilerParams(needs_layout_passes=False),
)

_init_k = pl.kernel(
    _init_body,
    out_type=(
        jax.ShapeDtypeStruct((N_PAD,), jnp.float32),
        jax.ShapeDtypeStruct((N_PAD,), jnp.float32),
        jax.ShapeDtypeStruct((N_PAD, D), jnp.float32),
    ),
    mesh=_MESH,
    scratch_types=[
        pltpu.VMEM((W_ROWS,), jnp.float32),
        pltpu.VMEM((W_ROWS,), jnp.float32),
        pltpu.VMEM((W_ROWS,), jnp.float32),
        pltpu.VMEM((W_ROWS,), jnp.float32),
        pltpu.VMEM((ROW_CHUNK, D), jnp.float32),
    ],
    compiler_params=pltpu.CompilerParams(needs_layout_passes=False),
)

_edge_k = pl.kernel(
    _edge_body,
    out_type=(
        jax.ShapeDtypeStruct((N_PAD, D), jnp.float32),
        jax.ShapeDtypeStruct((N_PAD, D), jnp.float32),
    ),

    mesh=_MESH,
    scratch_types=[
        pltpu.VMEM_SHARED((N_PAD, D), jnp.float32),
        pltpu.VMEM((BATCH_E,), jnp.int32),
        pltpu.VMEM((BATCH, EDGE_CHUNK), jnp.int32),
        pltpu.VMEM((EDGE_CHUNK, D), jnp.float32),
        pltpu.VMEM((EDGE_CHUNK, D), jnp.float32),
        pltpu.SemaphoreType.DMA,
        pltpu.SemaphoreType.DMA,
        pltpu.SemaphoreType.DMA,
        pltpu.SemaphoreType.DMA,
    ],
    compiler_params=pltpu.CompilerParams(needs_layout_passes=False),
)

_rescale_k = pl.kernel(
    _rescale_body,
    out_type=jax.ShapeDtypeStruct((N_PAD, D), jnp.float32),
    mesh=_MESH,
    scratch_types=[
        pltpu.VMEM((ROW_CHUNK, D), jnp.float32),
        pltpu.VMEM((ROW_CHUNK, D), jnp.float32),
        pltpu.VMEM((ROW_CHUNK, D), jnp.float32),
        pltpu.VMEM((W_ROWS,), jnp.float32),
    ],
    compiler_params=pltpu.CompilerParams(needs_layout_passes=False),
)


@jax.jit
def _propagate(x_pad, src_pad, dst2_pad):
    pdeg0, pdeg1 = _deg_k(dst2_pad)
    dinv, dinv2, g = _init_k(x_pad, pdeg0, pdeg1)
    for hop in range(K_HOPS):
        part0, part1 = _edge_k(g, src_pad, dst2_pad)
        scale = dinv if hop == K_HOPS - 1 else dinv2
        g = _rescale_k(part0, part1, g, scale)
    return g


def _linear_elu_body(h_ref, wt_ref, b_ref, o_ref):
    o = (jnp.dot(h_ref[:], wt_ref[:], preferred_element_type=jnp.float32)
         + b_ref[:])
    o_ref[:, :] = jnp.where(o > 0, o, jnp.exp(o) - jnp.float32(1.0))


@jax.jit
def _linear_elu(h2, wt, b2d):
    blk = 1024
    grid = (N_PAD // blk,)
    return pl.pallas_call(
        _linear_elu_body,
        grid=grid,
        in_specs=[
            pl.BlockSpec((blk, D), lambda i: (i, 0)),
            pl.BlockSpec((D, H), lambda i: (0, 0)),
            pl.BlockSpec((1, H), lambda i: (0, 0)),
        ],
        out_specs=pl.BlockSpec((blk, H), lambda i: (i, 0)),
        out_shape=jax.ShapeDtypeStruct((N_PAD, H), jnp.float32),
    )(h2, wt, b2d)


def kernel(x, edge_index, W, b):
    x_pad = jnp.zeros((N_PAD, D), jnp.float32).at[:N].set(x)
    pad_idx = jnp.full((E_PAD - E,), N_PAD - 1, jnp.int32)
    src_pad = jnp.concatenate([edge_index[0], pad_idx])
    dst_pad = jnp.concatenate([edge_index[1], pad_idx])
    dst2_pad = dst_pad.reshape(E_PAD // EDGE_CHUNK, EDGE_CHUNK)
    h2 = _propagate(x_pad, src_pad, dst2_pad)

    out = _linear_elu(h2, W.T, b[None, :])
    return out[:N]


# speedup vs baseline: 1.4714x; 1.0956x over previous
import jax
import jax.numpy as jnp
from jax import lax
from jax.experimental import pallas as pl
from jax.experimental.pallas import tpu as pltpu
from jax.experimental.pallas import tpu_sc as plsc

N = 10000
E = 320000
D = 128
H = 128
K_HOPS = 3

NC = 2
NS = 16
NW = NC * NS
LANES = 16

N_PAD = 10240
ROWS_PER_TILE = N_PAD // NS
W_ROWS = N_PAD // NW
ROW_CHUNK = 64
DEG_CHUNK = 128
N_DEG_CHUNKS = ROWS_PER_TILE // DEG_CHUNK

EDGE_CHUNK = 128
BATCH = 8
BATCH_E = BATCH * EDGE_CHUNK
B_FAST = 18
B_SLOW = 2
N_BATCHES_PAIR = B_FAST + B_SLOW
E_PAD = N_BATCHES_PAIR * BATCH_E * NS

_MESH = plsc.VectorSubcoreMesh(core_axis_name="c", subcore_axis_name="s")


def _rsqrt16(x):
    i = lax.bitcast_convert_type(x, jnp.int32)
    i = jnp.int32(0x5F3759DF) - lax.shift_right_arithmetic(i, 1)
    y = lax.bitcast_convert_type(i, jnp.float32)
    for _ in range(3):
        y = y * (jnp.float32(1.5) - jnp.float32(0.5) * x * y * y)
    return y


def _worker(c, s):
    return c * NS + s


def _deg_body(dst2_hbm, pdeg0, pdeg1,
              degstage_sh, deg_v, degred_v, idx_d):
    c = lax.axis_index("c")
    s = lax.axis_index("s")
    w = _worker(c, s)
    r0 = s * ROWS_PER_TILE
    dbase = w * ((E_PAD // NW) // EDGE_CHUNK)

    def zero_deg(i, _):
        deg_v[pl.ds(i * LANES, LANES)] = jnp.zeros((LANES,), jnp.float32)
        return _
    lax.fori_loop(0, N_PAD // LANES, zero_deg, 0)

    ones = jnp.ones((LANES,), jnp.float32)

    def deg_batch(ib, _):
        pltpu.sync_copy(dst2_hbm.at[pl.ds(dbase + ib * BATCH, BATCH)], idx_d)
        def deg_row(t, _):
            for q in range(EDGE_CHUNK // LANES):
                idx = idx_d[t, pl.ds(q * LANES, LANES)]
                plsc.addupdate_scatter(deg_v, [idx], ones)
            return _
        lax.fori_loop(0, BATCH, deg_row, 0)
        return _
    lax.fori_loop(0, E_PAD // NW // BATCH_E, deg_batch, 0)

    pltpu.sync_copy(deg_v, degstage_sh.at[s])
    plsc.subcore_barrier()

    def deg_reduce(j, _):
        pltpu.sync_copy(degstage_sh.at[:, pl.ds(r0 + j * DEG_CHUNK,
                                                DEG_CHUNK)], degred_v)
        for q in range(DEG_CHUNK // LANES):
            acc = jnp.zeros((LANES,), jnp.float32)
            for k in range(NS):
                acc = acc + degred_v[k, pl.ds(q * LANES, LANES)]
            deg_v[pl.ds(j * DEG_CHUNK + q * LANES, LANES)] = acc
        return _
    lax.fori_loop(0, N_DEG_CHUNKS, deg_reduce, 0)

    @pl.when(c == 0)
    def _dump0():
        pltpu.sync_copy(deg_v.at[pl.ds(0, ROWS_PER_TILE)],
                        pdeg0.at[pl.ds(r0, ROWS_PER_TILE)])

    @pl.when(c == 1)
    def _dump1():
        pltpu.sync_copy(deg_v.at[pl.ds(0, ROWS_PER_TILE)],
                        pdeg1.at[pl.ds(r0, ROWS_PER_TILE)])


def _init_body(x_hbm, pdeg0, pdeg1,
               dinv_out, dinv2_out, g0_out,
               d0_v, d1_v, dv_v, dv2_v, xbuf):
    c = lax.axis_index("c")
    s = lax.axis_index("s")
    r0 = _worker(c, s) * W_ROWS

    pltpu.sync_copy(pdeg0.at[pl.ds(r0, W_ROWS)], d0_v)
    pltpu.sync_copy(pdeg1.at[pl.ds(r0, W_ROWS)], d1_v)

    def dv_group(j, _):
        cs = pl.ds(j * LANES, LANES)
        dtot = d0_v[cs] + d1_v[cs] + jnp.float32(1.0)
        y = _rsqrt16(dtot)
        dv_v[cs] = y
        dv2_v[cs] = y * y
        return _
    lax.fori_loop(0, W_ROWS // LANES, dv_group, 0)

    pltpu.sync_copy(dv_v, dinv_out.at[pl.ds(r0, W_ROWS)])
    pltpu.sync_copy(dv2_v, dinv2_out.at[pl.ds(r0, W_ROWS)])

    def init_chunk(j, _):
        r = r0 + j * ROW_CHUNK
        pltpu.sync_copy(x_hbm.at[pl.ds(r, ROW_CHUNK)], xbuf)
        def g16_grp(g16, _):
            dvc = dv_v[pl.ds(j * ROW_CHUNK + g16 * LANES, LANES)]
            for lane in range(LANES):
                sc = dvc[lane]
                for q in range(D // LANES):
                    cs = pl.ds(q * LANES, LANES)
                    xbuf[g16 * LANES + lane, cs] = (
                        xbuf[g16 * LANES + lane, cs] * sc)
            return _
        lax.fori_loop(0, ROW_CHUNK // LANES, g16_grp, 0)
        pltpu.sync_copy(xbuf, g0_out.at[pl.ds(r, ROW_CHUNK)])
        return _
    lax.fori_loop(0, W_ROWS // ROW_CHUNK, init_chunk, 0)


def _edge_body(g_hbm, src_hbm, dst2_hbm,
               part0, part1,
               acc_sh, idx_s, idx_d, rows0, rows1,
               sem_g0, sem_g1, sem_s0, sem_s1):
    c = lax.axis_index("c")
    s = lax.axis_index("s")
    slab = s * ROWS_PER_TILE
    nb = jnp.where(c == 0, B_FAST, B_SLOW)
    bbase = jnp.where(c == 0, s * B_FAST, NS * B_FAST + s * B_SLOW)
    ebase = bbase * BATCH_E
    dbase = bbase * BATCH

    @pl.when(nb > 0)
    def _zero_acc():
        def zrow(i, _):
            for q in range(D // LANES):
                rows0[i, pl.ds(q * LANES, LANES)] = jnp.zeros((LANES,),
                                                              jnp.float32)
            return _
        lax.fori_loop(0, EDGE_CHUNK, zrow, 0)
        def zcopy(i, _):
            pltpu.sync_copy(rows0,
                            acc_sh.at[pl.ds(slab + i * EDGE_CHUNK,
                                            EDGE_CHUNK)])
            return _
        lax.fori_loop(0, ROWS_PER_TILE // EDGE_CHUNK, zcopy, 0)
    plsc.subcore_barrier()

    def g_start(buf, sem, j):
        pltpu.async_copy(
            g_hbm.at[idx_s.at[pl.ds(j * EDGE_CHUNK, EDGE_CHUNK)]], buf, sem)

    def g_wait(buf, sem):
        pltpu.make_async_copy(
            g_hbm.at[idx_s.at[pl.ds(0, EDGE_CHUNK)]], buf, sem).wait()

    def s_start(buf, sem, j):
        pltpu.async_copy(buf, acc_sh.at[idx_d.at[j]], sem, add=True)

    def s_wait(buf, sem):
        pltpu.make_async_copy(buf, acc_sh.at[idx_d.at[0]], sem).wait()

    def emit_batch(ib, _):
        pltpu.sync_copy(src_hbm.at[pl.ds(ebase + ib * BATCH_E, BATCH_E)],
                        idx_s)
        pltpu.sync_copy(dst2_hbm.at[pl.ds(dbase + ib * BATCH, BATCH)], idx_d)
        g_start(rows0, sem_g0, 0)
        g_wait(rows0, sem_g0)

        @pl.when(ib > 0)
        def _drain_prev():
            s_wait(rows1, sem_s1)
        s_start(rows0, sem_s0, 0)
        g_start(rows1, sem_g1, 1)

        def pair(k, _):
            c1 = 2 * k + 1
            c2 = 2 * k + 2
            g_wait(rows1, sem_g1)
            s_wait(rows0, sem_s0)
            s_start(rows1, sem_s1, c1)
            g_start(rows0, sem_g0, c2)
            g_wait(rows0, sem_g0)
            s_wait(rows1, sem_s1)
            s_start(rows0, sem_s0, c2)
            g_start(rows1, sem_g1, c2 + 1)
            return _
        lax.fori_loop(0, BATCH // 2 - 1, pair, 0)

        g_wait(rows1, sem_g1)
        s_wait(rows0, sem_s0)
        s_start(rows1, sem_s1, BATCH - 1)
        return _
    lax.fori_loop(0, nb, emit_batch, 0)

    @pl.when(nb > 0)
    def _drain_last():
        s_wait(rows1, sem_s1)
    plsc.subcore_barrier()

    @pl.when(c == 0)
    def _dump0():
        pltpu.sync_copy(acc_sh.at[pl.ds(slab, ROWS_PER_TILE)],
                        part0.at[pl.ds(slab, ROWS_PER_TILE)])

    @pl.when(c == 1)
    def _dump1():
        pltpu.sync_copy(acc_sh.at[pl.ds(slab, ROWS_PER_TILE)],
                        part1.at[pl.ds(slab, ROWS_PER_TILE)])


def _rescale_body(p0_hbm, p1_hbm, g_hbm, scale_hbm,
                  out_hbm,
                  b0, b1, b2, sc_v):
    c = lax.axis_index("c")
    s = lax.axis_index("s")
    r0 = _worker(c, s) * W_ROWS

    pltpu.sync_copy(scale_hbm.at[pl.ds(r0, W_ROWS)], sc_v)

    def resc_chunk(j, _):
        r = r0 + j * ROW_CHUNK
        pltpu.sync_copy(p0_hbm.at[pl.ds(r, ROW_CHUNK)], b0)
        pltpu.sync_copy(p1_hbm.at[pl.ds(r, ROW_CHUNK)], b1)
        pltpu.sync_copy(g_hbm.at[pl.ds(r, ROW_CHUNK)], b2)
        def g16_grp(g16, _):
            dvc = sc_v[pl.ds(j * ROW_CHUNK + g16 * LANES, LANES)]
            for lane in range(LANES):
                sc = dvc[lane]
                rr = g16 * LANES + lane
                for q in range(D // LANES):
                    cs = pl.ds(q * LANES, LANES)
                    b2[rr, cs] = (b2[rr, cs] + b0[rr, cs] + b1[rr, cs]) * sc
            return _
        lax.fori_loop(0, ROW_CHUNK // LANES, g16_grp, 0)
        pltpu.sync_copy(b2, out_hbm.at[pl.ds(r, ROW_CHUNK)])
        return _
    lax.fori_loop(0, W_ROWS // ROW_CHUNK, resc_chunk, 0)


_deg_k = pl.kernel(
    _deg_body,
    out_type=(
        jax.ShapeDtypeStruct((N_PAD,), jnp.float32),
        jax.ShapeDtypeStruct((N_PAD,), jnp.float32),
    ),
    mesh=_MESH,
    scratch_types=[
        pltpu.VMEM_SHARED((NS, N_PAD), jnp.float32),
        pltpu.VMEM((N_PAD,), jnp.float32),
        pltpu.VMEM((NS, DEG_CHUNK), jnp.float32),
        pltpu.VMEM((BATCH, EDGE_CHUNK), jnp.int32),
    ],
    compiler_params=pltpu.CompilerParams(needs_layout_passes=False),
)

_init_k = pl.kernel(
    _init_body,
    out_type=(
        jax.ShapeDtypeStruct((N_PAD,), jnp.float32),
        jax.ShapeDtypeStruct((N_PAD,), jnp.float32),
        jax.ShapeDtypeStruct((N_PAD, D), jnp.float32),
    ),
    mesh=_MESH,
    scratch_types=[
        pltpu.VMEM((W_ROWS,), jnp.float32),
        pltpu.VMEM((W_ROWS,), jnp.float32),
        pltpu.VMEM((W_ROWS,), jnp.float32),
        pltpu.VMEM((W_ROWS,), jnp.float32),
        pltpu.VMEM((ROW_CHUNK, D), jnp.float32),
    ],
    compiler_params=pltpu.CompilerParams(needs_layout_passes=False),
)

_edge_k = pl.kernel(
    _edge_body,
    out_type=(
        jax.ShapeDtypeStruct((N_PAD, D), jnp.float32),
        jax.ShapeDtypeStruct((N_PAD, D), jnp.float32),
    ),

    mesh=_MESH,
    scratch_types=[
        pltpu.VMEM_SHARED((N_PAD, D), jnp.float32),
        pltpu.VMEM((BATCH_E,), jnp.int32),
        pltpu.VMEM((BATCH, EDGE_CHUNK), jnp.int32),
        pltpu.VMEM((EDGE_CHUNK, D), jnp.float32),
        pltpu.VMEM((EDGE_CHUNK, D), jnp.float32),
        pltpu.SemaphoreType.DMA,
        pltpu.SemaphoreType.DMA,
        pltpu.SemaphoreType.DMA,
        pltpu.SemaphoreType.DMA,
    ],
    compiler_params=pltpu.CompilerParams(needs_layout_passes=False),
)

_rescale_k = pl.kernel(
    _rescale_body,
    out_type=jax.ShapeDtypeStruct((N_PAD, D), jnp.float32),
    mesh=_MESH,
    scratch_types=[
        pltpu.VMEM((ROW_CHUNK, D), jnp.float32),
        pltpu.VMEM((ROW_CHUNK, D), jnp.float32),
        pltpu.VMEM((ROW_CHUNK, D), jnp.float32),
        pltpu.VMEM((W_ROWS,), jnp.float32),
    ],
    compiler_params=pltpu.CompilerParams(needs_layout_passes=False),
)


@jax.jit
def _propagate(x_pad, src_pad, dst2_pad):
    pdeg0, pdeg1 = _deg_k(dst2_pad)
    dinv, dinv2, g = _init_k(x_pad, pdeg0, pdeg1)
    for hop in range(K_HOPS):
        part0, part1 = _edge_k(g, src_pad, dst2_pad)
        scale = dinv if hop == K_HOPS - 1 else dinv2
        g = _rescale_k(part0, part1, g, scale)
    return g


def _linear_elu_body(h_ref, wt_ref, b_ref, o_ref):
    o = (jnp.dot(h_ref[:], wt_ref[:], preferred_element_type=jnp.float32)
         + b_ref[:])
    o_ref[:, :] = jnp.where(o > 0, o, jnp.exp(o) - jnp.float32(1.0))


@jax.jit
def _linear_elu(h2, wt, b2d):
    blk = 1024
    grid = (N_PAD // blk,)
    return pl.pallas_call(
        _linear_elu_body,
        grid=grid,
        in_specs=[
            pl.BlockSpec((blk, D), lambda i: (i, 0)),
            pl.BlockSpec((D, H), lambda i: (0, 0)),
            pl.BlockSpec((1, H), lambda i: (0, 0)),
        ],
        out_specs=pl.BlockSpec((blk, H), lambda i: (i, 0)),
        out_shape=jax.ShapeDtypeStruct((N_PAD, H), jnp.float32),
    )(h2, wt, b2d)


def kernel(x, edge_index, W, b):
    x_pad = jnp.zeros((N_PAD, D), jnp.float32).at[:N].set(x)
    pad_idx = jnp.full((E_PAD - E,), N_PAD - 1, jnp.int32)
    src_pad = jnp.concatenate([edge_index[0], pad_idx])
    dst_pad = jnp.concatenate([edge_index[1], pad_idx])
    dst2_pad = dst_pad.reshape(E_PAD // EDGE_CHUNK, EDGE_CHUNK)
    h2 = _propagate(x_pad, src_pad, dst2_pad)

    out = _linear_elu(h2, W.T, b[None, :])
    return out[:N]
